# baseline (device time: 63784 ns/iter reference)
import jax
import jax.numpy as jnp
from jax import lax
from jax.experimental import pallas as pl
from jax.experimental.pallas import tpu as pltpu

N_DEV = 4


def _layer(x, win, wout, *, collective_id):
    b, d_shard = x.shape
    _, hdim = win.shape
    q = hdim // N_DEV

    def body(x_ref, win_hbm, wout_hbm, out_ref, win_v, wout_v, pbuf,
             rs_buf, h_buf, win_sems, wout_sem, rs_send, rs_recv,
             ag_send, ag_recv):
        my_pos = lax.axis_index("i")

        win_cp = []
        for j in range(N_DEV):
            peer = (my_pos + 1 + j) % N_DEV if j < N_DEV - 1 else my_pos
            cp = pltpu.make_async_copy(
                win_hbm.at[:, pl.ds(peer * q, q)],
                win_v.at[:, pl.ds(peer * q, q)],
                win_sems.at[j],
            )
            cp.start()
            win_cp.append(cp)
        wout_cp = pltpu.make_async_copy(wout_hbm, wout_v, wout_sem)
        wout_cp.start()

        barrier_sem = pltpu.get_barrier_semaphore()
        for k in range(1, N_DEV):
            pl.semaphore_signal(
                barrier_sem, inc=1,
                device_id=((my_pos + k) % N_DEV,),
                device_id_type=pl.DeviceIdType.MESH,
            )
        pl.semaphore_wait(barrier_sem, N_DEV - 1)

        xv = x_ref[...]

        rs = []
        for k in range(1, N_DEV):
            peer = (my_pos + k) % N_DEV
            win_cp[k - 1].wait()
            pbuf[k - 1] = jnp.dot(xv, win_v[:, pl.ds(peer * q, q)],
                                  preferred_element_type=jnp.float32)
            rdma = pltpu.make_async_remote_copy(
                src_ref=pbuf.at[k - 1],
                dst_ref=rs_buf.at[k - 1],
                send_sem=rs_send.at[k - 1],
                recv_sem=rs_recv.at[k - 1],
                device_id=(peer,),
                device_id_type=pl.DeviceIdType.MESH,
            )
            rdma.start()
            rs.append(rdma)
        win_cp[N_DEV - 1].wait()
        own_q = jnp.dot(xv, win_v[:, pl.ds(my_pos * q, q)],
                        preferred_element_type=jnp.float32)
        for rdma in rs:
            rdma.wait()

        hq = jnp.maximum(own_q + rs_buf[0] + rs_buf[1] + rs_buf[2], 0.0)
        h_buf[:, pl.ds(my_pos * q, q)] = hq

        ag = []
        for k in range(1, N_DEV):
            peer = (my_pos + k) % N_DEV
            rdma = pltpu.make_async_remote_copy(
                src_ref=h_buf.at[:, pl.ds(my_pos * q, q)],
                dst_ref=h_buf.at[:, pl.ds(my_pos * q, q)],
                send_sem=ag_send.at[k - 1],
                recv_sem=ag_recv.at[k - 1],
                device_id=(peer,),
                device_id_type=pl.DeviceIdType.MESH,
            )
            rdma.start()
            ag.append(rdma)

        wout_cp.wait()
        acc = jnp.dot(hq, wout_v[pl.ds(my_pos * q, q), :],
                      preferred_element_type=jnp.float32)
        for rdma in ag:
            rdma.wait()
        for k in range(1, N_DEV):
            peer = (my_pos + k) % N_DEV
            acc = acc + jnp.dot(h_buf[:, pl.ds(peer * q, q)],
                                wout_v[pl.ds(peer * q, q), :],
                                preferred_element_type=jnp.float32)
        out_ref[...] = acc

    return pl.pallas_call(
        body,
        out_shape=jax.ShapeDtypeStruct((b, d_shard), jnp.float32),
        in_specs=[
            pl.BlockSpec(memory_space=pltpu.VMEM),
            pl.BlockSpec(memory_space=pl.ANY),
            pl.BlockSpec(memory_space=pl.ANY),
        ],
        out_specs=pl.BlockSpec(memory_space=pltpu.VMEM),
        scratch_shapes=[
            pltpu.VMEM((d_shard, hdim), jnp.float32),
            pltpu.VMEM((hdim, d_shard), jnp.float32),
            pltpu.VMEM((N_DEV - 1, b, q), jnp.float32),
            pltpu.VMEM((N_DEV - 1, b, q), jnp.float32),
            pltpu.VMEM((b, hdim), jnp.float32),
            pltpu.SemaphoreType.DMA((N_DEV,)),
            pltpu.SemaphoreType.DMA,
            pltpu.SemaphoreType.DMA((N_DEV - 1,)),
            pltpu.SemaphoreType.DMA((N_DEV - 1,)),
            pltpu.SemaphoreType.DMA((N_DEV - 1,)),
            pltpu.SemaphoreType.DMA((N_DEV - 1,)),
        ],
        compiler_params=pltpu.CompilerParams(collective_id=collective_id),
    )(x, win, wout)


def kernel(x, Win0, Wout0, Win1, Wout1, Win2, Wout2):
    x = _layer(x, Win0, Wout0, collective_id=0)
    x = _layer(x, Win1, Wout1, collective_id=1)
    x = _layer(x, Win2, Wout2, collective_id=2)
    return x


# device time: 57280 ns/iter; 1.1135x vs baseline; 1.1135x over previous
import jax
import jax.numpy as jnp
from jax import lax
from jax.experimental import pallas as pl
from jax.experimental.pallas import tpu as pltpu

N_DEV = 4


def _layer(x, win, wout, *, collective_id):
    b, d_shard = x.shape
    _, hdim = win.shape
    q = hdim // N_DEV

    def body(x_ref, win_ref, wout_hbm, out_ref, wout_v, pbuf,
             rs_buf, h_buf, wout_sem, rs_send, rs_recv,
             ag_send, ag_recv):
        my_pos = lax.axis_index("i")

        wout_cp = pltpu.make_async_copy(wout_hbm, wout_v, wout_sem)
        wout_cp.start()

        barrier_sem = pltpu.get_barrier_semaphore()
        for k in range(1, N_DEV):
            pl.semaphore_signal(
                barrier_sem, inc=1,
                device_id=((my_pos + k) % N_DEV,),
                device_id_type=pl.DeviceIdType.MESH,
            )
        pl.semaphore_wait(barrier_sem, N_DEV - 1)

        xv = x_ref[...]

        rs = []
        for k in range(1, N_DEV):
            peer = (my_pos + k) % N_DEV
            pbuf[k - 1] = jnp.dot(xv, win_ref[:, pl.ds(peer * q, q)],
                                  preferred_element_type=jnp.float32)
            rdma = pltpu.make_async_remote_copy(
                src_ref=pbuf.at[k - 1],
                dst_ref=rs_buf.at[k - 1],
                send_sem=rs_send.at[k - 1],
                recv_sem=rs_recv.at[k - 1],
                device_id=(peer,),
                device_id_type=pl.DeviceIdType.MESH,
            )
            rdma.start()
            rs.append(rdma)
        own_q = jnp.dot(xv, win_ref[:, pl.ds(my_pos * q, q)],
                        preferred_element_type=jnp.float32)
        for rdma in rs:
            rdma.wait()

        hq = jnp.maximum(own_q + rs_buf[0] + rs_buf[1] + rs_buf[2], 0.0)
        h_buf[:, pl.ds(my_pos * q, q)] = hq

        ag = []
        for k in range(1, N_DEV):
            peer = (my_pos + k) % N_DEV
            rdma = pltpu.make_async_remote_copy(
                src_ref=h_buf.at[:, pl.ds(my_pos * q, q)],
                dst_ref=h_buf.at[:, pl.ds(my_pos * q, q)],
                send_sem=ag_send.at[k - 1],
                recv_sem=ag_recv.at[k - 1],
                device_id=(peer,),
                device_id_type=pl.DeviceIdType.MESH,
            )
            rdma.start()
            ag.append(rdma)

        wout_cp.wait()
        acc = jnp.dot(hq, wout_v[pl.ds(my_pos * q, q), :],
                      preferred_element_type=jnp.float32)
        for rdma in ag:
            rdma.wait()
        for k in range(1, N_DEV):
            peer = (my_pos + k) % N_DEV
            acc = acc + jnp.dot(h_buf[:, pl.ds(peer * q, q)],
                                wout_v[pl.ds(peer * q, q), :],
                                preferred_element_type=jnp.float32)
        out_ref[...] = acc

    return pl.pallas_call(
        body,
        out_shape=jax.ShapeDtypeStruct((b, d_shard), jnp.float32),
        in_specs=[
            pl.BlockSpec(memory_space=pltpu.VMEM),
            pl.BlockSpec(memory_space=pltpu.VMEM),
            pl.BlockSpec(memory_space=pl.ANY),
        ],
        out_specs=pl.BlockSpec(memory_space=pltpu.VMEM),
        scratch_shapes=[
            pltpu.VMEM((hdim, d_shard), jnp.float32),
            pltpu.VMEM((N_DEV - 1, b, q), jnp.float32),
            pltpu.VMEM((N_DEV - 1, b, q), jnp.float32),
            pltpu.VMEM((b, hdim), jnp.float32),
            pltpu.SemaphoreType.DMA,
            pltpu.SemaphoreType.DMA((N_DEV - 1,)),
            pltpu.SemaphoreType.DMA((N_DEV - 1,)),
            pltpu.SemaphoreType.DMA((N_DEV - 1,)),
            pltpu.SemaphoreType.DMA((N_DEV - 1,)),
        ],
        compiler_params=pltpu.CompilerParams(collective_id=collective_id),
    )(x, win, wout)


def kernel(x, Win0, Wout0, Win1, Wout1, Win2, Wout2):
    x = _layer(x, Win0, Wout0, collective_id=0)
    x = _layer(x, Win1, Wout1, collective_id=1)
    x = _layer(x, Win2, Wout2, collective_id=2)
    return x


# device time: 57218 ns/iter; 1.1148x vs baseline; 1.0011x over previous
import jax
import jax.numpy as jnp
from jax import lax
from jax.experimental import pallas as pl
from jax.experimental.pallas import tpu as pltpu

N_DEV = 4


def _layer(x, win, wout, *, collective_id):
    b, d_shard = x.shape
    _, hdim = win.shape
    q = hdim // N_DEV

    def body(x_ref, win_ref, wout_hbm, out_ref, wout_v, pbuf,
             rs_buf, h_buf, wout_sem, rs_send, rs_recv,
             ag_send, ag_recv):
        my_pos = lax.axis_index("i")

        wout_cp = pltpu.make_async_copy(wout_hbm, wout_v, wout_sem)
        wout_cp.start()

        barrier_sem = pltpu.get_barrier_semaphore()
        for k in range(1, N_DEV):
            pl.semaphore_signal(
                barrier_sem, inc=1,
                device_id=((my_pos + k) % N_DEV,),
                device_id_type=pl.DeviceIdType.MESH,
            )
        pl.semaphore_wait(barrier_sem, N_DEV - 1)

        xv = x_ref[...]

        rs = []
        for k in range(1, N_DEV):
            peer = (my_pos + k) % N_DEV
            pbuf[k - 1] = jnp.dot(xv, win_ref[:, pl.ds(peer * q, q)],
                                  preferred_element_type=jnp.float32)
            rdma = pltpu.make_async_remote_copy(
                src_ref=pbuf.at[k - 1],
                dst_ref=rs_buf.at[k - 1],
                send_sem=rs_send.at[k - 1],
                recv_sem=rs_recv.at[k - 1],
                device_id=(peer,),
                device_id_type=pl.DeviceIdType.MESH,
            )
            rdma.start()
            rs.append(rdma)
        own_q = jnp.dot(xv, win_ref[:, pl.ds(my_pos * q, q)],
                        preferred_element_type=jnp.float32)
        for rdma in rs:
            rdma.wait()

        hq = jnp.maximum(own_q + rs_buf[0] + rs_buf[1] + rs_buf[2], 0.0)
        h_buf[:, pl.ds(my_pos * q, q)] = hq

        ag = []
        for k in range(1, N_DEV):
            peer = (my_pos + k) % N_DEV
            rdma = pltpu.make_async_remote_copy(
                src_ref=h_buf.at[:, pl.ds(my_pos * q, q)],
                dst_ref=h_buf.at[:, pl.ds(my_pos * q, q)],
                send_sem=ag_send.at[k - 1],
                recv_sem=ag_recv.at[k - 1],
                device_id=(peer,),
                device_id_type=pl.DeviceIdType.MESH,
            )
            rdma.start()
            ag.append(rdma)

        wout_cp.wait()
        acc = jnp.dot(hq, wout_v[pl.ds(my_pos * q, q), :],
                      preferred_element_type=jnp.float32)
        for rdma in ag:
            rdma.wait()
        for k in range(1, N_DEV):
            peer = (my_pos + k) % N_DEV
            acc = acc + jnp.dot(h_buf[:, pl.ds(peer * q, q)],
                                wout_v[pl.ds(peer * q, q), :],
                                preferred_element_type=jnp.float32)
        out_ref[...] = acc

    return pl.pallas_call(
        body,
        out_shape=jax.ShapeDtypeStruct((b, d_shard), jnp.float32),
        in_specs=[
            pl.BlockSpec(memory_space=pltpu.VMEM),
            pl.BlockSpec(memory_space=pltpu.VMEM),
            pl.BlockSpec(memory_space=pltpu.MemorySpace.HBM),
        ],
        out_specs=pl.BlockSpec(memory_space=pltpu.VMEM),
        scratch_shapes=[
            pltpu.VMEM((hdim, d_shard), jnp.float32),
            pltpu.VMEM((N_DEV - 1, b, q), jnp.float32),
            pltpu.VMEM((N_DEV - 1, b, q), jnp.float32),
            pltpu.VMEM((b, hdim), jnp.float32),
            pltpu.SemaphoreType.DMA,
            pltpu.SemaphoreType.DMA((N_DEV - 1,)),
            pltpu.SemaphoreType.DMA((N_DEV - 1,)),
            pltpu.SemaphoreType.DMA((N_DEV - 1,)),
            pltpu.SemaphoreType.DMA((N_DEV - 1,)),
        ],
        compiler_params=pltpu.CompilerParams(collective_id=collective_id),
    )(x, win, wout)


def kernel(x, Win0, Wout0, Win1, Wout1, Win2, Wout2):
    x = _layer(x, Win0, Wout0, collective_id=0)
    x = _layer(x, Win1, Wout1, collective_id=1)
    x = _layer(x, Win2, Wout2, collective_id=2)
    return x


# device time: 45122 ns/iter; 1.4136x vs baseline; 1.2681x over previous
import jax
import jax.numpy as jnp
from jax import lax
from jax.experimental import pallas as pl
from jax.experimental.pallas import tpu as pltpu

N_DEV = 4


def _layer(x, win, wout, *, collective_id):
    b, d_shard = x.shape
    _, hdim = win.shape
    q = hdim // N_DEV

    def body(x_ref, win_ref, wout_hbm, out_ref, wout_v, pbuf,
             rs_buf, h_buf, wout_sem, rs_send, rs_recv,
             ag_send, ag_recv):
        my_pos = lax.axis_index("i")

        wout_cp = pltpu.make_async_copy(wout_hbm, wout_v, wout_sem)
        wout_cp.start()

        barrier_sem = pltpu.get_barrier_semaphore()
        for k in range(1, N_DEV):
            pl.semaphore_signal(
                barrier_sem, inc=1,
                device_id=((my_pos + k) % N_DEV,),
                device_id_type=pl.DeviceIdType.MESH,
            )
        pl.semaphore_wait(barrier_sem, N_DEV - 1)

        xv = x_ref[...]

        rs = []
        for k in range(1, N_DEV):
            peer = (my_pos + k) % N_DEV
            pbuf[k - 1] = jnp.dot(xv, win_ref[:, pl.ds(peer * q, q)],
                                  preferred_element_type=jnp.float32)
            rdma = pltpu.make_async_remote_copy(
                src_ref=pbuf.at[k - 1],
                dst_ref=rs_buf.at[k - 1],
                send_sem=rs_send.at[k - 1],
                recv_sem=rs_recv.at[k - 1],
                device_id=(peer,),
                device_id_type=pl.DeviceIdType.MESH,
            )
            rdma.start()
            rs.append(rdma)
        own_q = jnp.dot(xv, win_ref[:, pl.ds(my_pos * q, q)],
                        preferred_element_type=jnp.float32)
        for rdma in rs:
            rdma.wait()

        hq = jnp.maximum(own_q + rs_buf[0] + rs_buf[1] + rs_buf[2], 0.0)
        h_buf[:, pl.ds(my_pos * q, q)] = hq

        ag = []
        for k in range(1, N_DEV):
            peer = (my_pos + k) % N_DEV
            rdma = pltpu.make_async_remote_copy(
                src_ref=h_buf.at[:, pl.ds(my_pos * q, q)],
                dst_ref=h_buf.at[:, pl.ds(my_pos * q, q)],
                send_sem=ag_send.at[k - 1],
                recv_sem=ag_recv.at[k - 1],
                device_id=(peer,),
                device_id_type=pl.DeviceIdType.MESH,
            )
            rdma.start()
            ag.append(rdma)

        wout_cp.wait()
        acc = jnp.dot(hq, wout_v[pl.ds(my_pos * q, q), :],
                      preferred_element_type=jnp.float32)
        for rdma in ag:
            rdma.wait()
        for k in range(1, N_DEV):
            peer = (my_pos + k) % N_DEV
            acc = acc + jnp.dot(h_buf[:, pl.ds(peer * q, q)],
                                wout_v[pl.ds(peer * q, q), :],
                                preferred_element_type=jnp.float32)
        out_ref[...] = acc

    return pl.pallas_call(
        body,
        out_shape=jax.ShapeDtypeStruct((b, d_shard), jnp.float32),
        in_specs=[
            pl.BlockSpec(memory_space=pltpu.VMEM),
            pl.BlockSpec(memory_space=pltpu.VMEM),
            pl.BlockSpec(memory_space=pltpu.MemorySpace.HBM),
        ],
        out_specs=pl.BlockSpec(memory_space=pltpu.VMEM),
        scratch_shapes=[
            pltpu.VMEM((hdim, d_shard), jnp.float32),
            pltpu.VMEM((N_DEV - 1, b, q), jnp.float32),
            pltpu.VMEM((N_DEV - 1, b, q), jnp.float32),
            pltpu.VMEM((b, hdim), jnp.float32),
            pltpu.SemaphoreType.DMA,
            pltpu.SemaphoreType.DMA((N_DEV - 1,)),
            pltpu.SemaphoreType.DMA((N_DEV - 1,)),
            pltpu.SemaphoreType.DMA((N_DEV - 1,)),
            pltpu.SemaphoreType.DMA((N_DEV - 1,)),
        ],
        compiler_params=pltpu.CompilerParams(collective_id=collective_id),
    )(x, win, pltpu.with_memory_space_constraint(
        wout, pltpu.MemorySpace.HBM))


def kernel(x, Win0, Wout0, Win1, Wout1, Win2, Wout2):
    x = _layer(x, Win0, Wout0, collective_id=0)
    x = _layer(x, Win1, Wout1, collective_id=1)
    x = _layer(x, Win2, Wout2, collective_id=2)
    return x


# device time: 34348 ns/iter; 1.8570x vs baseline; 1.3137x over previous
import jax
import jax.numpy as jnp
from jax import lax
from jax.experimental import pallas as pl
from jax.experimental.pallas import tpu as pltpu

N_DEV = 4
ARRIVAL_ORDER = (1, 3, 2)


def _layer(x, win, wout, *, collective_id):
    b, d_shard = x.shape
    _, hdim = win.shape
    q = hdim // N_DEV

    def body(x_ref, win_ref, wout_hbm, out_ref, wout_v, pbuf,
             rs_buf, h_buf, wout_sem, rs_send, rs_recv,
             ag_send, ag_recv):
        my_pos = lax.axis_index("i")

        wout_cp = pltpu.make_async_copy(wout_hbm, wout_v, wout_sem)
        wout_cp.start()

        barrier_sem = pltpu.get_barrier_semaphore()
        for k in range(1, N_DEV):
            pl.semaphore_signal(
                barrier_sem, inc=1,
                device_id=((my_pos + k) % N_DEV,),
                device_id_type=pl.DeviceIdType.MESH,
            )
        pl.semaphore_wait(barrier_sem, N_DEV - 1)

        xv = x_ref[...]

        rs = {}
        for k in range(1, N_DEV):
            peer = (my_pos + k) % N_DEV
            pbuf[k - 1] = jnp.dot(
                xv, win_ref[:, pl.ds(peer * q, q)],
                preferred_element_type=jnp.float32).astype(jnp.bfloat16)
            rdma = pltpu.make_async_remote_copy(
                src_ref=pbuf.at[k - 1],
                dst_ref=rs_buf.at[k - 1],
                send_sem=rs_send.at[k - 1],
                recv_sem=rs_recv.at[k - 1],
                device_id=(peer,),
                device_id_type=pl.DeviceIdType.MESH,
            )
            rdma.start()
            rs[k] = rdma
        own_q = jnp.dot(xv, win_ref[:, pl.ds(my_pos * q, q)],
                        preferred_element_type=jnp.float32)

        hq = own_q
        for k in ARRIVAL_ORDER:
            rs[k].wait()
            hq = hq + rs_buf[k - 1].astype(jnp.float32)
        hq = jnp.maximum(hq, 0.0)
        h_buf[:, pl.ds(my_pos * q, q)] = hq.astype(jnp.bfloat16)

        ag = {}
        for k in range(1, N_DEV):
            peer = (my_pos + k) % N_DEV
            rdma = pltpu.make_async_remote_copy(
                src_ref=h_buf.at[:, pl.ds(my_pos * q, q)],
                dst_ref=h_buf.at[:, pl.ds(my_pos * q, q)],
                send_sem=ag_send.at[k - 1],
                recv_sem=ag_recv.at[k - 1],
                device_id=(peer,),
                device_id_type=pl.DeviceIdType.MESH,
            )
            rdma.start()
            ag[k] = rdma

        wout_cp.wait()
        acc = jnp.dot(hq, wout_v[pl.ds(my_pos * q, q), :],
                      preferred_element_type=jnp.float32)
        for k in ARRIVAL_ORDER:
            ag[k].wait()
            src = (my_pos - k) % N_DEV
            acc = acc + jnp.dot(
                h_buf[:, pl.ds(src * q, q)].astype(jnp.float32),
                wout_v[pl.ds(src * q, q), :],
                preferred_element_type=jnp.float32)
        out_ref[...] = acc

    return pl.pallas_call(
        body,
        out_shape=jax.ShapeDtypeStruct((b, d_shard), jnp.float32),
        in_specs=[
            pl.BlockSpec(memory_space=pltpu.VMEM),
            pl.BlockSpec(memory_space=pltpu.VMEM),
            pl.BlockSpec(memory_space=pltpu.MemorySpace.HBM),
        ],
        out_specs=pl.BlockSpec(memory_space=pltpu.VMEM),
        scratch_shapes=[
            pltpu.VMEM((hdim, d_shard), jnp.float32),
            pltpu.VMEM((N_DEV - 1, b, q), jnp.bfloat16),
            pltpu.VMEM((N_DEV - 1, b, q), jnp.bfloat16),
            pltpu.VMEM((b, hdim), jnp.bfloat16),
            pltpu.SemaphoreType.DMA,
            pltpu.SemaphoreType.DMA((N_DEV - 1,)),
            pltpu.SemaphoreType.DMA((N_DEV - 1,)),
            pltpu.SemaphoreType.DMA((N_DEV - 1,)),
            pltpu.SemaphoreType.DMA((N_DEV - 1,)),
        ],
        compiler_params=pltpu.CompilerParams(collective_id=collective_id),
    )(x, win, pltpu.with_memory_space_constraint(
        wout, pltpu.MemorySpace.HBM))


def kernel(x, Win0, Wout0, Win1, Wout1, Win2, Wout2):
    x = _layer(x, Win0, Wout0, collective_id=0)
    x = _layer(x, Win1, Wout1, collective_id=1)
    x = _layer(x, Win2, Wout2, collective_id=2)
    return x
